# baseline (speedup 1.0000x reference)
import functools

import jax
import jax.numpy as jnp
from jax import lax
from jax.experimental import pallas as pl
from jax.experimental.pallas import tpu as pltpu
from jax.experimental.pallas import tpu_sc as plsc

N = 10000
E = 160000
D = 128
NPG = 100
NB = 100
NPROT = 3
NPN = 100

NC, NS, K = 2, 16, 64
NW = NC * NS
MCH = 80
EPAD = NW * MCH * K
RTR = N + 240
OPT = RTR // NS

_F32 = jnp.float32
_SDS = jax.ShapeDtypeStruct

def _edge_agg_body(hh, src3, dst3, zc, outc,
                   src2_v, dst2_v, ra0, ra1, c_sh, gs0, gs1, ss0, ss1):
    cid = lax.axis_index("c")
    sid = lax.axis_index("s")
    wid = sid * NC + cid
    base = sid * OPT

    for j in range(OPT // K):
        pltpu.sync_copy(zc.at[pl.ds(base + j * K, K)], ra0)
        pltpu.sync_copy(ra0, c_sh.at[pl.ds(base + j * K, K)])
    pltpu.sync_copy(src3.at[wid], src2_v)
    pltpu.sync_copy(dst3.at[wid], dst2_v)
    plsc.subcore_barrier()

    npair = MCH // 2
    pltpu.async_copy(hh.at[src2_v.at[0]], ra0, gs0)

    def _pair(p, carry):
        j0 = 2 * p
        pltpu.make_async_copy(hh.at[src2_v.at[j0]], ra0, gs0).wait()

        @pl.when(p > 0)
        def _():
            pltpu.make_async_copy(ra1, c_sh.at[dst2_v.at[j0 - 1]], ss1).wait()

        pltpu.async_copy(hh.at[src2_v.at[j0 + 1]], ra1, gs1)
        pltpu.async_copy(ra0, c_sh.at[dst2_v.at[j0]], ss0, add=True)
        pltpu.make_async_copy(hh.at[src2_v.at[j0 + 1]], ra1, gs1).wait()
        pltpu.make_async_copy(ra0, c_sh.at[dst2_v.at[j0]], ss0).wait()

        @pl.when(p < npair - 1)
        def _():
            pltpu.async_copy(hh.at[src2_v.at[j0 + 2]], ra0, gs0)

        pltpu.async_copy(ra1, c_sh.at[dst2_v.at[j0 + 1]], ss1, add=True)
        return carry

    lax.fori_loop(0, npair, _pair, 0)
    pltpu.make_async_copy(ra1, c_sh.at[dst2_v.at[MCH - 1]], ss1).wait()
    plsc.subcore_barrier()

    for j in range(OPT // K):
        pltpu.sync_copy(c_sh.at[pl.ds(base + j * K, K)], ra0)
        pltpu.sync_copy(ra0, outc.at[cid, pl.ds(base + j * K, K)])


@functools.cache
def _make_edge_agg():
    mesh = plsc.VectorSubcoreMesh(
        core_axis_name="c", subcore_axis_name="s",
        num_cores=NC, num_subcores=NS)
    return pl.kernel(
        _edge_agg_body,
        out_type=_SDS((NC, RTR, D), _F32),
        mesh=mesh,
        scratch_types=[
            pltpu.VMEM((MCH, K), jnp.int32),
            pltpu.VMEM((MCH, K), jnp.int32),
            pltpu.VMEM((K, D), _F32),
            pltpu.VMEM((K, D), _F32),
            pltpu.VMEM_SHARED((RTR, D), _F32),
            pltpu.SemaphoreType.DMA,
            pltpu.SemaphoreType.DMA,
            pltpu.SemaphoreType.DMA,
            pltpu.SemaphoreType.DMA,
        ],
    )


def _tc(body, out_shape, *args):
    return pl.pallas_call(
        body,
        out_shape=out_shape,
        compiler_params=pltpu.CompilerParams(
            vmem_limit_bytes=100 * 1024 * 1024),
    )(*args)


def _embed_body(h_ref, w_ref, b_ref, o_ref):
    o_ref[...] = jnp.dot(h_ref[...], w_ref[...],
                         preferred_element_type=_F32) + b_ref[...]


def _sage_body(h_ref, c0_ref, c1_ref, d0_ref, d1_ref, w_ref, b_ref, g_ref,
               bt_ref, o_ref):
    h = h_ref[...]
    deg = jnp.maximum(d0_ref[...] + d1_ref[...], 1.0)
    c = (c0_ref[...] + c1_ref[...]) / deg
    bundle = (jnp.dot(h, w_ref[0:D, :], preferred_element_type=_F32)
              + jnp.dot(c, w_ref[D:2 * D, :], preferred_element_type=_F32)
              + b_ref[...])
    nrm = jnp.maximum(jnp.sqrt(jnp.sum(bundle * bundle, axis=1,
                                       keepdims=True)), 1e-12)
    hn = jnp.maximum(bundle / nrm, 0.0)
    mu = jnp.mean(hn, axis=0, keepdims=True)
    var = jnp.mean((hn - mu) ** 2, axis=0, keepdims=True)
    o_ref[...] = h + g_ref[...] * (hn - mu) / jnp.sqrt(var + 1e-5) + bt_ref[...]


def _recon_body(h_ref, w0, b0, w1, b1, w2, b2, wd, o_x, o_l, o_r, o_hg):
    hv = h_ref[...]
    a0 = jnp.maximum(jnp.dot(hv, w0[...], preferred_element_type=_F32)
                     + b0[...], 0.0)
    a1 = jnp.maximum(jnp.dot(a0, w1[...], preferred_element_type=_F32)
                     + b1[...], 0.0)
    x = jnp.dot(a1, w2[...], preferred_element_type=_F32) + b2[...]
    o_x[...] = x
    o_l[...] = jnp.dot(x, wd[0:D, :], preferred_element_type=_F32)
    o_r[...] = jnp.dot(x, wd[D:2 * D, :], preferred_element_type=_F32)
    row = lax.broadcasted_iota(jnp.int32, (NB, N), 0)
    col = lax.broadcasted_iota(jnp.int32, (NB, N), 1) // NPG
    gmat = jnp.where(row == col, 1.0 / NPG, 0.0).astype(_F32)
    o_hg[...] = jnp.dot(gmat, hv, preferred_element_type=_F32)


def _outer_body(l_ref, r2_ref, b_ref, o_ref):
    row = lax.broadcasted_iota(jnp.int32, (N, NB), 0) // NPG
    col = lax.broadcasted_iota(jnp.int32, (N, NB), 1)
    gexp = jnp.where(row == col, 1.0, 0.0).astype(_F32)
    rb = jnp.dot(gexp, r2_ref[...], preferred_element_type=_F32)
    z = l_ref[...] + rb + b_ref[...]
    o_ref[...] = 1.0 / (1.0 + jnp.exp(-z))


def _proto_body(p_ref, a_ref, hg_ref, we, be, sw0, sb0, sg0, sbt0,
                sw1, sb1, sg1, sbt1, w0, b0, w1, b1, w2, b2, wd, bd, o_ref):
    eye = jnp.where(
        lax.broadcasted_iota(jnp.int32, (NPN, NPN), 0)
        == lax.broadcasted_iota(jnp.int32, (NPN, NPN), 1), 1.0, 0.0)
    ones_col = jnp.full((NPN, 1), 1.0, _F32)
    sparams = ((sw0, sb0, sg0, sbt0), (sw1, sb1, sg1, sbt1))
    hgv = hg_ref[...]
    dists = []
    for k in range(2 * NPROT):
        e = p_ref[k]
        a_p = a_ref[k]
        a0 = jnp.maximum(jnp.dot(e, w0[...], preferred_element_type=_F32)
                         + b0[...], 0.0)
        a1 = jnp.maximum(jnp.dot(a0, w1[...], preferred_element_type=_F32)
                         + b1[...], 0.0)
        x = jnp.dot(a1, w2[...], preferred_element_type=_F32) + b2[...]
        l = jnp.dot(x, wd[0:D, :], preferred_element_type=_F32)
        r = jnp.dot(x, wd[D:2 * D, :], preferred_element_type=_F32)
        rrow = lax.dot_general(r, eye, (((0,), (0,)), ((), ())))
        s = 1.0 / (1.0 + jnp.exp(-(l + rrow + bd[...])))
        th = a_p * 0.2 + (1.0 - a_p) * 0.8
        ab = jnp.where(s > th, 1.0, 0.0).astype(_F32)
        deg_col = jnp.maximum(
            lax.dot_general(ab, ones_col, (((0,), (0,)), ((), ()))), 1.0)
        hp = jnp.dot(x, we[...], preferred_element_type=_F32) + be[...]
        for (W, bb, g, bt) in sparams:
            c = lax.dot_general(ab, hp, (((0,), (0,)), ((), ()))) / deg_col
            bundle = (jnp.dot(hp, W[0:D, :], preferred_element_type=_F32)
                      + jnp.dot(c, W[D:2 * D, :], preferred_element_type=_F32)
                      + bb[...])
            nrm = jnp.maximum(jnp.sqrt(jnp.sum(bundle * bundle, axis=1,
                                               keepdims=True)), 1e-12)
            hn = jnp.maximum(bundle / nrm, 0.0)
            mu = jnp.mean(hn, axis=0, keepdims=True)
            var = jnp.mean((hn - mu) ** 2, axis=0, keepdims=True)
            hp = hp + g[...] * (hn - mu) / jnp.sqrt(var + 1e-5) + bt[...]
        hk = jnp.mean(hp, axis=0, keepdims=True)
        diff = hgv - hk
        dists.append(jnp.sum(diff * diff, axis=1, keepdims=True))
    dist = jnp.concatenate(dists, axis=1)
    ss = jnp.log((dist + 1.0) / (dist + 1e-12))
    m = jnp.max(ss, axis=1, keepdims=True)
    ex = jnp.exp(ss - m)
    w_ = ex / jnp.sum(ex, axis=1, keepdims=True)
    colidx = lax.broadcasted_iota(jnp.int32, (NB, 2 * NPROT), 1)
    o_ref[...] = jnp.sum(jnp.where(colidx >= NPROT, w_, 0.0), axis=1,
                         keepdims=True)


def _deg_body(dst3, zc, oc, outd, dst2_v, buf_v, d_sh, sem):
    cid = lax.axis_index("c")
    sid = lax.axis_index("s")
    wid = sid * NC + cid
    base = sid * OPT

    for j in range(OPT // K):
        pltpu.sync_copy(zc.at[pl.ds(base + j * K, K)], buf_v)
        pltpu.sync_copy(buf_v, d_sh.at[pl.ds(base + j * K, K)])
    pltpu.sync_copy(oc, buf_v)
    pltpu.sync_copy(dst3.at[wid], dst2_v)
    plsc.subcore_barrier()

    BF = 8

    def _batch(b, carry):
        j0 = b * BF
        for i in range(BF):
            pltpu.async_copy(buf_v, d_sh.at[dst2_v.at[j0 + i]], sem,
                             add=True)
        for i in range(BF):
            pltpu.make_async_copy(buf_v, d_sh.at[dst2_v.at[j0 + i]],
                                  sem).wait()
        return carry

    lax.fori_loop(0, MCH // BF, _batch, 0)
    plsc.subcore_barrier()

    for j in range(OPT // K):
        pltpu.sync_copy(d_sh.at[pl.ds(base + j * K, K)], buf_v)
        pltpu.sync_copy(buf_v, outd.at[cid, pl.ds(base + j * K, K)])


@functools.cache
def _make_deg():
    mesh = plsc.VectorSubcoreMesh(
        core_axis_name="c", subcore_axis_name="s",
        num_cores=NC, num_subcores=NS)
    return pl.kernel(
        _deg_body,
        out_type=_SDS((NC, RTR, D), _F32),
        mesh=mesh,
        scratch_types=[
            pltpu.VMEM((MCH, K), jnp.int32),
            pltpu.VMEM((K, D), _F32),
            pltpu.VMEM_SHARED((RTR, D), _F32),
            pltpu.SemaphoreType.DMA,
        ],
    )


def _edge_partials(hh, src3, dst3, zc):
    return _make_edge_agg()(hh, src3, dst3, zc)


def _deg_partials(dst3, zc, oc):
    return _make_deg()(dst3, zc, oc)


def kernel(h, edge_index, W_emb, b_emb, sW0, sb0, sg0, sbt0, sW1, sb1, sg1,
           sbt1, dW0, db0, dW1, db1, dW2, db2, W_dec2, b_dec2, p_neg, p_pos,
           a_neg, a_pos):
    src = edge_index[0].astype(jnp.int32)
    dst = edge_index[1].astype(jnp.int32)
    npad = EPAD - E
    src3 = jnp.concatenate([src, jnp.zeros((npad,), jnp.int32)]
                           ).reshape(NW, MCH, K)
    dst3 = jnp.concatenate([dst, jnp.full((npad,), N, jnp.int32)]
                           ).reshape(NW, MCH, K)
    zc = jnp.zeros((RTR, D), _F32)
    oc = jnp.ones((K, D), _F32)

    be = b_emb.reshape(1, D)
    sb0r, sg0r, sbt0r = sb0.reshape(1, D), sg0.reshape(1, D), sbt0.reshape(1, D)
    sb1r, sg1r, sbt1r = sb1.reshape(1, D), sg1.reshape(1, D), sbt1.reshape(1, D)
    db0r, db1r, db2r = db0.reshape(1, -1), db1.reshape(1, -1), db2.reshape(1, -1)
    bd = b_dec2.reshape(1, 1)

    hh = _tc(_embed_body, _SDS((N, D), _F32), h, W_emb, be)

    dparts = _deg_partials(dst3, zc, oc)
    d0, d1 = dparts[0, :N, 0:1], dparts[1, :N, 0:1]

    cparts = _edge_partials(hh, src3, dst3, zc)
    hh = _tc(_sage_body, _SDS((N, D), _F32), hh, cparts[0, :N], cparts[1, :N],
             d0, d1, sW0, sb0r, sg0r, sbt0r)

    cparts2 = _edge_partials(hh, src3, dst3, zc)
    hh = _tc(_sage_body, _SDS((N, D), _F32), hh, cparts2[0, :N], cparts2[1, :N],
             d0, d1, sW1, sb1r, sg1r, sbt1r)

    x, l, r, hg = _tc(
        _recon_body,
        (_SDS((N, D), _F32), _SDS((N, 1), _F32), _SDS((N, 1), _F32),
         _SDS((NB, D), _F32)),
        hh, dW0, db0r, dW1, db1r, dW2, db2r, W_dec2)

    s2 = _tc(_outer_body, _SDS((N, NB), _F32), l, r.reshape(NB, NPG), bd)

    protos = jnp.concatenate([p_neg, p_pos], axis=0)
    adjs = jnp.concatenate([a_neg, a_pos], axis=0)
    out2 = _tc(_proto_body, _SDS((NB, 1), _F32), protos, adjs, hg,
               W_emb, be, sW0, sb0r, sg0r, sbt0r, sW1, sb1r, sg1r, sbt1r,
               dW0, db0r, dW1, db1r, dW2, db2r, W_dec2, bd)

    return (out2.reshape(NB), x.reshape(NB, NPG, D),
            s2.reshape(NB, NPG, NPG))


# speedup vs baseline: 1.5436x; 1.5436x over previous
import functools

import jax
import jax.numpy as jnp
from jax import lax
from jax.experimental import pallas as pl
from jax.experimental.pallas import tpu as pltpu
from jax.experimental.pallas import tpu_sc as plsc

N = 10000
E = 160000
D = 128
NPG = 100
NB = 100
NPROT = 3
NPN = 100

NC, NS, K = 2, 16, 64
NW = NC * NS
MCH = 80
EPAD = NW * MCH * K
M0, M1 = 116, 42
MMAX = M0
ECAP = NS * K * (M0 + M1)
RTR = N + 240
OPT = RTR // NS

_F32 = jnp.float32
_SDS = jax.ShapeDtypeStruct

def _edge_agg_body(hh, src3, dst3, zc, outc,
                   src2_v, dst2_v, ra0, ra1, c_sh, gs0, gs1, ss0, ss1):
    cid = lax.axis_index("c")
    sid = lax.axis_index("s")
    base = sid * OPT

    for j in range(OPT // K):
        pltpu.sync_copy(zc.at[pl.ds(base + j * K, K)], ra0)
        pltpu.sync_copy(ra0, c_sh.at[pl.ds(base + j * K, K)])
    pltpu.sync_copy(src3.at[cid, sid], src2_v)
    pltpu.sync_copy(dst3.at[cid, sid], dst2_v)
    plsc.subcore_barrier()

    npair = jnp.where(cid == 0, M0 // 2, M1 // 2)
    mlast = jnp.where(cid == 0, M0 - 1, M1 - 1)
    pltpu.async_copy(hh.at[src2_v.at[0]], ra0, gs0)

    def _pair(p, carry):
        j0 = 2 * p
        pltpu.make_async_copy(hh.at[src2_v.at[j0]], ra0, gs0).wait()

        @pl.when(p > 0)
        def _():
            pltpu.make_async_copy(ra1, c_sh.at[dst2_v.at[j0 - 1]], ss1).wait()

        pltpu.async_copy(hh.at[src2_v.at[j0 + 1]], ra1, gs1)
        pltpu.async_copy(ra0, c_sh.at[dst2_v.at[j0]], ss0, add=True)
        pltpu.make_async_copy(hh.at[src2_v.at[j0 + 1]], ra1, gs1).wait()
        pltpu.make_async_copy(ra0, c_sh.at[dst2_v.at[j0]], ss0).wait()

        @pl.when(p < npair - 1)
        def _():
            pltpu.async_copy(hh.at[src2_v.at[j0 + 2]], ra0, gs0)

        pltpu.async_copy(ra1, c_sh.at[dst2_v.at[j0 + 1]], ss1, add=True)
        return carry

    lax.fori_loop(0, npair, _pair, 0)
    pltpu.make_async_copy(ra1, c_sh.at[dst2_v.at[mlast]], ss1).wait()
    plsc.subcore_barrier()

    for j in range(OPT // K):
        pltpu.sync_copy(c_sh.at[pl.ds(base + j * K, K)], ra0)
        pltpu.sync_copy(ra0, outc.at[cid, pl.ds(base + j * K, K)])


@functools.cache
def _make_edge_agg():
    mesh = plsc.VectorSubcoreMesh(
        core_axis_name="c", subcore_axis_name="s",
        num_cores=NC, num_subcores=NS)
    return pl.kernel(
        _edge_agg_body,
        out_type=_SDS((NC, RTR, D), _F32),
        mesh=mesh,
        scratch_types=[
            pltpu.VMEM((MMAX, K), jnp.int32),
            pltpu.VMEM((MMAX, K), jnp.int32),
            pltpu.VMEM((K, D), _F32),
            pltpu.VMEM((K, D), _F32),
            pltpu.VMEM_SHARED((RTR, D), _F32),
            pltpu.SemaphoreType.DMA,
            pltpu.SemaphoreType.DMA,
            pltpu.SemaphoreType.DMA,
            pltpu.SemaphoreType.DMA,
        ],
    )


def _tc(body, out_shape, *args):
    return pl.pallas_call(
        body,
        out_shape=out_shape,
        compiler_params=pltpu.CompilerParams(
            vmem_limit_bytes=100 * 1024 * 1024),
    )(*args)


def _embed_body(h_ref, w_ref, b_ref, o_ref):
    o_ref[...] = jnp.dot(h_ref[...], w_ref[...],
                         preferred_element_type=_F32) + b_ref[...]


def _sage_body(h_ref, c0_ref, c1_ref, d0_ref, d1_ref, w_ref, b_ref, g_ref,
               bt_ref, o_ref):
    h = h_ref[...]
    deg = jnp.maximum(d0_ref[...] + d1_ref[...], 1.0)
    c = (c0_ref[...] + c1_ref[...]) / deg
    bundle = (jnp.dot(h, w_ref[0:D, :], preferred_element_type=_F32)
              + jnp.dot(c, w_ref[D:2 * D, :], preferred_element_type=_F32)
              + b_ref[...])
    nrm = jnp.maximum(jnp.sqrt(jnp.sum(bundle * bundle, axis=1,
                                       keepdims=True)), 1e-12)
    hn = jnp.maximum(bundle / nrm, 0.0)
    mu = jnp.mean(hn, axis=0, keepdims=True)
    var = jnp.mean((hn - mu) ** 2, axis=0, keepdims=True)
    o_ref[...] = h + g_ref[...] * (hn - mu) / jnp.sqrt(var + 1e-5) + bt_ref[...]


def _recon_body(h_ref, w0, b0, w1, b1, w2, b2, wd, o_x, o_l, o_r, o_hg):
    hv = h_ref[...]
    a0 = jnp.maximum(jnp.dot(hv, w0[...], preferred_element_type=_F32)
                     + b0[...], 0.0)
    a1 = jnp.maximum(jnp.dot(a0, w1[...], preferred_element_type=_F32)
                     + b1[...], 0.0)
    x = jnp.dot(a1, w2[...], preferred_element_type=_F32) + b2[...]
    o_x[...] = x
    o_l[...] = jnp.dot(x, wd[0:D, :], preferred_element_type=_F32)
    o_r[...] = jnp.dot(x, wd[D:2 * D, :], preferred_element_type=_F32)
    row = lax.broadcasted_iota(jnp.int32, (NB, N), 0)
    col = lax.broadcasted_iota(jnp.int32, (NB, N), 1) // NPG
    gmat = jnp.where(row == col, 1.0 / NPG, 0.0).astype(_F32)
    o_hg[...] = jnp.dot(gmat, hv, preferred_element_type=_F32)


def _outer_body(l_ref, r2_ref, b_ref, o_ref):
    row = lax.broadcasted_iota(jnp.int32, (N, NB), 0) // NPG
    col = lax.broadcasted_iota(jnp.int32, (N, NB), 1)
    gexp = jnp.where(row == col, 1.0, 0.0).astype(_F32)
    rb = jnp.dot(gexp, r2_ref[...], preferred_element_type=_F32)
    z = l_ref[...] + rb + b_ref[...]
    o_ref[...] = 1.0 / (1.0 + jnp.exp(-z))


def _proto_body(p_ref, a_ref, hg_ref, we, be, sw0, sb0, sg0, sbt0,
                sw1, sb1, sg1, sbt1, w0, b0, w1, b1, w2, b2, wd, bd, o_ref):
    eye = jnp.where(
        lax.broadcasted_iota(jnp.int32, (NPN, NPN), 0)
        == lax.broadcasted_iota(jnp.int32, (NPN, NPN), 1), 1.0, 0.0)
    ones_col = jnp.full((NPN, 1), 1.0, _F32)
    sparams = ((sw0, sb0, sg0, sbt0), (sw1, sb1, sg1, sbt1))
    hgv = hg_ref[...]
    dists = []
    for k in range(2 * NPROT):
        e = p_ref[k]
        a_p = a_ref[k]
        a0 = jnp.maximum(jnp.dot(e, w0[...], preferred_element_type=_F32)
                         + b0[...], 0.0)
        a1 = jnp.maximum(jnp.dot(a0, w1[...], preferred_element_type=_F32)
                         + b1[...], 0.0)
        x = jnp.dot(a1, w2[...], preferred_element_type=_F32) + b2[...]
        l = jnp.dot(x, wd[0:D, :], preferred_element_type=_F32)
        r = jnp.dot(x, wd[D:2 * D, :], preferred_element_type=_F32)
        rrow = lax.dot_general(r, eye, (((0,), (0,)), ((), ())))
        s = 1.0 / (1.0 + jnp.exp(-(l + rrow + bd[...])))
        th = a_p * 0.2 + (1.0 - a_p) * 0.8
        ab = jnp.where(s > th, 1.0, 0.0).astype(_F32)
        deg_col = jnp.maximum(
            lax.dot_general(ab, ones_col, (((0,), (0,)), ((), ()))), 1.0)
        hp = jnp.dot(x, we[...], preferred_element_type=_F32) + be[...]
        for (W, bb, g, bt) in sparams:
            c = lax.dot_general(ab, hp, (((0,), (0,)), ((), ()))) / deg_col
            bundle = (jnp.dot(hp, W[0:D, :], preferred_element_type=_F32)
                      + jnp.dot(c, W[D:2 * D, :], preferred_element_type=_F32)
                      + bb[...])
            nrm = jnp.maximum(jnp.sqrt(jnp.sum(bundle * bundle, axis=1,
                                               keepdims=True)), 1e-12)
            hn = jnp.maximum(bundle / nrm, 0.0)
            mu = jnp.mean(hn, axis=0, keepdims=True)
            var = jnp.mean((hn - mu) ** 2, axis=0, keepdims=True)
            hp = hp + g[...] * (hn - mu) / jnp.sqrt(var + 1e-5) + bt[...]
        hk = jnp.mean(hp, axis=0, keepdims=True)
        diff = hgv - hk
        dists.append(jnp.sum(diff * diff, axis=1, keepdims=True))
    dist = jnp.concatenate(dists, axis=1)
    ss = jnp.log((dist + 1.0) / (dist + 1e-12))
    m = jnp.max(ss, axis=1, keepdims=True)
    ex = jnp.exp(ss - m)
    w_ = ex / jnp.sum(ex, axis=1, keepdims=True)
    colidx = lax.broadcasted_iota(jnp.int32, (NB, 2 * NPROT), 1)
    o_ref[...] = jnp.sum(jnp.where(colidx >= NPROT, w_, 0.0), axis=1,
                         keepdims=True)


def _deg_body(dst3, zc, oc, outd, dst2_v, buf_v, d_sh, sem):
    cid = lax.axis_index("c")
    sid = lax.axis_index("s")
    wid = sid * NC + cid
    base = sid * OPT

    for j in range(OPT // K):
        pltpu.sync_copy(zc.at[pl.ds(base + j * K, K)], buf_v)
        pltpu.sync_copy(buf_v, d_sh.at[pl.ds(base + j * K, K)])
    pltpu.sync_copy(oc, buf_v)
    pltpu.sync_copy(dst3.at[wid], dst2_v)
    plsc.subcore_barrier()

    BF = 8

    def _batch(b, carry):
        j0 = b * BF
        for i in range(BF):
            pltpu.async_copy(buf_v, d_sh.at[dst2_v.at[j0 + i]], sem,
                             add=True)
        for i in range(BF):
            pltpu.make_async_copy(buf_v, d_sh.at[dst2_v.at[j0 + i]],
                                  sem).wait()
        return carry

    lax.fori_loop(0, MCH // BF, _batch, 0)
    plsc.subcore_barrier()

    for j in range(OPT // K):
        pltpu.sync_copy(d_sh.at[pl.ds(base + j * K, K)], buf_v)
        pltpu.sync_copy(buf_v, outd.at[cid, pl.ds(base + j * K, K)])


@functools.cache
def _make_deg():
    mesh = plsc.VectorSubcoreMesh(
        core_axis_name="c", subcore_axis_name="s",
        num_cores=NC, num_subcores=NS)
    return pl.kernel(
        _deg_body,
        out_type=_SDS((NC, RTR, D), _F32),
        mesh=mesh,
        scratch_types=[
            pltpu.VMEM((MCH, K), jnp.int32),
            pltpu.VMEM((K, D), _F32),
            pltpu.VMEM_SHARED((RTR, D), _F32),
            pltpu.SemaphoreType.DMA,
        ],
    )


def _edge_partials(hh, src3, dst3, zc):
    return _make_edge_agg()(hh, src3, dst3, zc)


def _deg_partials(dst3, zc, oc):
    return _make_deg()(dst3, zc, oc)


def kernel(h, edge_index, W_emb, b_emb, sW0, sb0, sg0, sbt0, sW1, sb1, sg1,
           sbt1, dW0, db0, dW1, db1, dW2, db2, W_dec2, b_dec2, p_neg, p_pos,
           a_neg, a_pos):
    src = edge_index[0].astype(jnp.int32)
    dst = edge_index[1].astype(jnp.int32)

    def _asym(e, fill):
        flat = jnp.concatenate(
            [e, jnp.full((ECAP - E,), fill, jnp.int32)])
        cut = NS * M0 * K
        b0 = flat[:cut].reshape(NS, M0, K)
        b1 = flat[cut:].reshape(NS, M1, K)
        b0 = jnp.pad(b0, ((0, 0), (0, MMAX - M0), (0, 0)),
                     constant_values=fill)
        b1 = jnp.pad(b1, ((0, 0), (0, MMAX - M1), (0, 0)),
                     constant_values=fill)
        return jnp.stack([b0, b1])

    src3 = _asym(src, 0)
    dst3 = _asym(dst, N)
    npad = EPAD - E
    dst3s = jnp.concatenate([dst, jnp.full((npad,), N, jnp.int32)]
                            ).reshape(NW, MCH, K)
    zc = jnp.zeros((RTR, D), _F32)
    oc = jnp.ones((K, D), _F32)

    be = b_emb.reshape(1, D)
    sb0r, sg0r, sbt0r = sb0.reshape(1, D), sg0.reshape(1, D), sbt0.reshape(1, D)
    sb1r, sg1r, sbt1r = sb1.reshape(1, D), sg1.reshape(1, D), sbt1.reshape(1, D)
    db0r, db1r, db2r = db0.reshape(1, -1), db1.reshape(1, -1), db2.reshape(1, -1)
    bd = b_dec2.reshape(1, 1)

    hh = _tc(_embed_body, _SDS((N, D), _F32), h, W_emb, be)

    dparts = _deg_partials(dst3s, zc, oc)
    d0, d1 = dparts[0, :N, 0:1], dparts[1, :N, 0:1]

    cparts = _edge_partials(hh, src3, dst3, zc)
    hh = _tc(_sage_body, _SDS((N, D), _F32), hh, cparts[0, :N], cparts[1, :N],
             d0, d1, sW0, sb0r, sg0r, sbt0r)

    cparts2 = _edge_partials(hh, src3, dst3, zc)
    hh = _tc(_sage_body, _SDS((N, D), _F32), hh, cparts2[0, :N], cparts2[1, :N],
             d0, d1, sW1, sb1r, sg1r, sbt1r)

    x, l, r, hg = _tc(
        _recon_body,
        (_SDS((N, D), _F32), _SDS((N, 1), _F32), _SDS((N, 1), _F32),
         _SDS((NB, D), _F32)),
        hh, dW0, db0r, dW1, db1r, dW2, db2r, W_dec2)

    s2 = _tc(_outer_body, _SDS((N, NB), _F32), l, r.reshape(NB, NPG), bd)

    protos = jnp.concatenate([p_neg, p_pos], axis=0)
    adjs = jnp.concatenate([a_neg, a_pos], axis=0)
    out2 = _tc(_proto_body, _SDS((NB, 1), _F32), protos, adjs, hg,
               W_emb, be, sW0, sb0r, sg0r, sbt0r, sW1, sb1r, sg1r, sbt1r,
               dW0, db0r, dW1, db1r, dW2, db2r, W_dec2, bd)

    return (out2.reshape(NB), x.reshape(NB, NPG, D),
            s2.reshape(NB, NPG, NPG))
